# baseline (speedup 1.0000x reference)
import jax
import jax.numpy as jnp
from jax.experimental import pallas as pl
from jax.experimental.pallas import tpu as pltpu

_BB = 2


def _prelu_body(route_ref, w_ref, f_ref, x_ref, o_ref):
    j = pl.program_id(0)
    for k in range(_BB):
        idx = route_ref[j * _BB + k]
        slope = (w_ref[idx] + f_ref[0])[None, :]
        xv = x_ref[k]
        o_ref[k] = jnp.where(xv >= 0, xv, slope * xv)


def kernel(input, route_index, weight, weight_fact):
    B, C, H, W = input.shape
    HW = H * W
    routes = route_index.astype(jnp.int32)
    x3 = input.transpose(0, 2, 3, 1).reshape(B, HW, C)

    grid_spec = pltpu.PrefetchScalarGridSpec(
        num_scalar_prefetch=1,
        grid=(B // _BB,),
        in_specs=[
            pl.BlockSpec(memory_space=pltpu.VMEM),
            pl.BlockSpec(memory_space=pltpu.VMEM),
            pl.BlockSpec((_BB, HW, C), lambda j, r: (j, 0, 0)),
        ],
        out_specs=pl.BlockSpec((_BB, HW, C), lambda j, r: (j, 0, 0)),
    )
    out = pl.pallas_call(
        _prelu_body,
        grid_spec=grid_spec,
        out_shape=jax.ShapeDtypeStruct((B, HW, C), jnp.float32),
        compiler_params=pltpu.CompilerParams(
            dimension_semantics=("parallel",),
        ),
    )(routes, weight, weight_fact, x3)
    return out.reshape(B, H, W, C).transpose(0, 3, 1, 2)
